# baseline (speedup 1.0000x reference)
import functools

import jax
import jax.numpy as jnp
from jax import lax
from jax.experimental import pallas as pl
from jax.experimental.pallas import tpu as pltpu
from jax.experimental.pallas import tpu_sc as plsc

_NUM_CORES = 2
_NUM_SUBCORES = 16
_NW = _NUM_CORES * _NUM_SUBCORES
_LANES = 16


def _sc_gather(table_flat, idx2, H, nrd):
    spad, W = idx2.shape
    tpad = table_flat.shape[0]
    quarter = tpad // 4
    nvec = W // _LANES
    main_rows = 16 * _NW
    rest = spad - min(spad, main_rows)
    assert rest % 8 == 0 and rest // 8 <= _NW
    mesh = plsc.VectorSubcoreMesh(core_axis_name="c", subcore_axis_name="s")

    @functools.partial(
        pl.kernel,
        out_type=jax.ShapeDtypeStruct((H, spad, W), jnp.float32),
        mesh=mesh,
        compiler_params=pltpu.CompilerParams(needs_layout_passes=False),
        scratch_types=[
            pltpu.VMEM((16, W), jnp.int32),
            pltpu.VMEM((tpad,), jnp.float32),
            [pltpu.VMEM((16, W), jnp.float32) for _ in range(4)],
            [pltpu.SemaphoreType.DMA for _ in range(4)],
        ],
    )
    def k(table_hbm, idx_hbm, out_hbm, idx_v, tab_v, bufs, sems):
        wid = lax.axis_index("s") * _NUM_CORES + lax.axis_index("c")
        for c in range(4):
            off = lax.rem(wid + c, 4) * quarter
            pltpu.sync_copy(table_hbm.at[pl.ds(off, quarter)],
                            tab_v.at[pl.ds(off, quarter)])
        nbuf = len(bufs)

        def do_span(r0, R):
            pltpu.sync_copy(idx_hbm.at[pl.ds(r0, R), :], idx_v.at[pl.ds(0, R), :])
            pending = [None] * nbuf
            for h in range(H):
                buf, sem = bufs[h % nbuf], sems[h % nbuf]
                if pending[h % nbuf] is not None:
                    pending[h % nbuf].wait()
                hoff = jnp.full((_LANES,), h * nrd, jnp.int32)

                def g_row(r, c):
                    def g_vec(j, c2):
                        sl = pl.ds(j * _LANES, _LANES)
                        buf[r, sl] = plsc.load_gather(
                            tab_v, [idx_v[r, sl] + hoff]
                        )
                        return c2

                    return lax.fori_loop(0, nvec, g_vec, c)

                lax.fori_loop(0, R, g_row, 0)
                pending[h % nbuf] = pltpu.async_copy(
                    buf.at[pl.ds(0, R), :],
                    out_hbm.at[h, pl.ds(r0, R), :],
                    sem,
                )
            for p in pending:
                if p is not None:
                    p.wait()

        if main_rows <= spad:
            do_span(wid * 16, 16)
        else:
            @pl.when(wid * 16 < spad)
            def _():
                do_span(wid * 16, 16)
        if rest:

            @pl.when(wid < rest // 8)
            def _():
                do_span(main_rows + wid * 8, 8)

    return k(table_flat, idx2)


def _tc_add(inputs, pos3):
    B, H, S1, S2 = inputs.shape
    _, spad, W = pos3.shape

    def body(x_ref, p_ref, o_ref):
        o_ref[:, 0] = x_ref[:, 0] + p_ref[:1, :S1, :S2]

    return pl.pallas_call(
        body,
        grid=(H,),
        in_specs=[
            pl.BlockSpec((B, 1, S1, S2), lambda h: (0, h, 0, 0)),
            pl.BlockSpec((1, spad, W), lambda h: (h, 0, 0)),
        ],
        out_specs=pl.BlockSpec((B, 1, S1, S2), lambda h: (0, h, 0, 0)),
        out_shape=jax.ShapeDtypeStruct((B, H, S1, S2), jnp.float32),
    )(inputs, pos3)


def kernel(inputs, positional_embedding, relative_position_index):
    B, H, S1, S2 = inputs.shape
    idx = relative_position_index[:S1, :S2]
    nrd = positional_embedding.shape[1]
    spad = -(-S1 // 8) * 8
    W = -(-S2 // _LANES) * _LANES
    idx2 = jnp.pad(idx.astype(jnp.int32), ((0, spad - S1), (0, W - S2)))
    tpad = -(-(H * nrd) // 32) * 32
    table_flat = jnp.pad(
        jnp.reshape(positional_embedding, (H * nrd,)), (0, tpad - H * nrd)
    )
    pos3 = _sc_gather(table_flat, idx2, H, nrd)
    return _tc_add(inputs, pos3)


# speedup vs baseline: 1.0503x; 1.0503x over previous
import functools

import jax
import jax.numpy as jnp
from jax import lax
from jax.experimental import pallas as pl
from jax.experimental.pallas import tpu as pltpu
from jax.experimental.pallas import tpu_sc as plsc

_NUM_CORES = 2
_NUM_SUBCORES = 16
_NW = _NUM_CORES * _NUM_SUBCORES
_LANES = 16


def _sc_gather(table_flat, idx2, H, nrd):
    spad, W = idx2.shape
    nvec = W // _LANES
    main_rows = 16 * _NW
    rest = spad - min(spad, main_rows)
    assert rest % 8 == 0 and rest // 8 <= _NW
    mesh = plsc.VectorSubcoreMesh(core_axis_name="c", subcore_axis_name="s")

    @functools.partial(
        pl.kernel,
        out_type=jax.ShapeDtypeStruct((H, spad, W), jnp.float32),
        mesh=mesh,
        compiler_params=pltpu.CompilerParams(needs_layout_passes=False),
        scratch_types=[
            pltpu.VMEM((16, W), jnp.int32),
            pltpu.VMEM((H * nrd,), jnp.float32),
            [pltpu.VMEM((16, W), jnp.float32) for _ in range(4)],
            [pltpu.SemaphoreType.DMA for _ in range(4)],
        ],
    )
    def k(table_hbm, idx_hbm, out_hbm, idx_v, tab_v, bufs, sems):
        wid = lax.axis_index("s") * _NUM_CORES + lax.axis_index("c")
        pltpu.sync_copy(table_hbm, tab_v)
        nbuf = len(bufs)

        def do_span(r0, R):
            pltpu.sync_copy(idx_hbm.at[pl.ds(r0, R), :], idx_v.at[pl.ds(0, R), :])
            pending = [None] * nbuf
            for h in range(H):
                buf, sem = bufs[h % nbuf], sems[h % nbuf]
                if pending[h % nbuf] is not None:
                    pending[h % nbuf].wait()
                hoff = jnp.full((_LANES,), h * nrd, jnp.int32)

                def g_row(r, c):
                    def g_vec(j, c2):
                        sl = pl.ds(j * _LANES, _LANES)
                        buf[r, sl] = plsc.load_gather(
                            tab_v, [idx_v[r, sl] + hoff]
                        )
                        return c2

                    return lax.fori_loop(0, nvec, g_vec, c)

                lax.fori_loop(0, R, g_row, 0)
                pending[h % nbuf] = pltpu.async_copy(
                    buf.at[pl.ds(0, R), :],
                    out_hbm.at[h, pl.ds(r0, R), :],
                    sem,
                )
            for p in pending:
                if p is not None:
                    p.wait()

        if main_rows <= spad:
            do_span(wid * 16, 16)
        else:
            @pl.when(wid * 16 < spad)
            def _():
                do_span(wid * 16, 16)
        if rest:

            @pl.when(wid < rest // 8)
            def _():
                do_span(main_rows + wid * 8, 8)

    return k(table_flat, idx2)


def _tc_add(inputs, pos3):
    B, H, S1, S2 = inputs.shape
    _, spad, W = pos3.shape

    def body(x_ref, p_ref, o_ref):
        o_ref[:, 0] = x_ref[:, 0] + p_ref[:1, :S1, :S2]

    return pl.pallas_call(
        body,
        grid=(H,),
        in_specs=[
            pl.BlockSpec((B, 1, S1, S2), lambda h: (0, h, 0, 0)),
            pl.BlockSpec((1, spad, W), lambda h: (h, 0, 0)),
        ],
        out_specs=pl.BlockSpec((B, 1, S1, S2), lambda h: (0, h, 0, 0)),
        out_shape=jax.ShapeDtypeStruct((B, H, S1, S2), jnp.float32),
    )(inputs, pos3)


def kernel(inputs, positional_embedding, relative_position_index):
    B, H, S1, S2 = inputs.shape
    idx = relative_position_index[:S1, :S2]
    nrd = positional_embedding.shape[1]
    spad = -(-S1 // 8) * 8
    W = -(-S2 // _LANES) * _LANES
    idx2 = jnp.pad(idx.astype(jnp.int32), ((0, spad - S1), (0, W - S2)))
    table_flat = jnp.reshape(positional_embedding, (H * nrd,))
    pos3 = _sc_gather(table_flat, idx2, H, nrd)
    return _tc_add(inputs, pos3)


# speedup vs baseline: 1.0531x; 1.0027x over previous
import functools

import jax
import jax.numpy as jnp
from jax import lax
from jax.experimental import pallas as pl
from jax.experimental.pallas import tpu as pltpu
from jax.experimental.pallas import tpu_sc as plsc

_NUM_CORES = 2
_NUM_SUBCORES = 16
_NW = _NUM_CORES * _NUM_SUBCORES
_LANES = 16


def _sc_gather(table_flat, idx2, H, nrd):
    spad, W = idx2.shape
    nvec = W // _LANES
    main_rows = 16 * _NW
    rest = spad - min(spad, main_rows)
    assert rest % 8 == 0 and rest // 8 <= _NW
    mesh = plsc.VectorSubcoreMesh(core_axis_name="c", subcore_axis_name="s")

    @functools.partial(
        pl.kernel,
        out_type=jax.ShapeDtypeStruct((H, spad, W), jnp.float32),
        mesh=mesh,
        compiler_params=pltpu.CompilerParams(needs_layout_passes=False),
        scratch_types=[
            pltpu.VMEM((16, W), jnp.int32),
            pltpu.VMEM((H * nrd,), jnp.float32),
            [pltpu.VMEM((16, W), jnp.float32) for _ in range(4)],
            [pltpu.SemaphoreType.DMA for _ in range(4)],
        ],
    )
    def k(table_hbm, idx_hbm, out_hbm, idx_v, tab_v, bufs, sems):
        wid = lax.axis_index("s") * _NUM_CORES + lax.axis_index("c")
        pltpu.sync_copy(table_hbm, tab_v)
        nbuf = len(bufs)

        def do_span(r0, R):
            pltpu.sync_copy(idx_hbm.at[pl.ds(r0, R), :], idx_v.at[pl.ds(0, R), :])
            pending = [None] * nbuf
            for h in range(H):
                buf, sem = bufs[h % nbuf], sems[h % nbuf]
                if pending[h % nbuf] is not None:
                    pending[h % nbuf].wait()
                hoff = jnp.full((_LANES,), h * nrd, jnp.int32)

                def g_row(r, c):
                    def g_vec(j, c2):
                        sl = pl.ds(j * _LANES, _LANES)
                        buf[r, sl] = plsc.load_gather(
                            tab_v, [idx_v[r, sl] + hoff]
                        )
                        return c2

                    return lax.fori_loop(0, nvec, g_vec, c)

                lax.fori_loop(0, R, g_row, 0)
                pending[h % nbuf] = pltpu.async_copy(
                    buf.at[pl.ds(0, R), :],
                    out_hbm.at[h, pl.ds(r0, R), :],
                    sem,
                )
            for p in pending:
                if p is not None:
                    p.wait()

        if main_rows <= spad:
            do_span(wid * 16, 16)
        else:
            @pl.when(wid * 16 < spad)
            def _():
                do_span(wid * 16, 16)
        if rest:

            @pl.when(wid < rest // 8)
            def _():
                do_span(main_rows + wid * 8, 8)

    return k(table_flat, idx2)


def _tc_add(inputs, pos3):
    B, H, S1, S2 = inputs.shape
    _, spad, W = pos3.shape

    HB = 2

    def body(x_ref, p_ref, o_ref):
        o_ref[...] = x_ref[...] + p_ref[:, :S1, :S2][None]

    return pl.pallas_call(
        body,
        grid=(H // HB,),
        in_specs=[
            pl.BlockSpec((B, HB, S1, S2), lambda h: (0, h, 0, 0)),
            pl.BlockSpec((HB, spad, W), lambda h: (h, 0, 0)),
        ],
        out_specs=pl.BlockSpec((B, HB, S1, S2), lambda h: (0, h, 0, 0)),
        out_shape=jax.ShapeDtypeStruct((B, H, S1, S2), jnp.float32),
    )(inputs, pos3)


def kernel(inputs, positional_embedding, relative_position_index):
    B, H, S1, S2 = inputs.shape
    idx = relative_position_index[:S1, :S2]
    nrd = positional_embedding.shape[1]
    spad = -(-S1 // 8) * 8
    W = -(-S2 // _LANES) * _LANES
    idx2 = jnp.pad(idx.astype(jnp.int32), ((0, spad - S1), (0, W - S2)))
    table_flat = jnp.reshape(positional_embedding, (H * nrd,))
    pos3 = _sc_gather(table_flat, idx2, H, nrd)
    return _tc_add(inputs, pos3)
